# baseline (speedup 1.0000x reference)
import functools

import jax
import jax.numpy as jnp
from jax import lax
from jax.experimental import pallas as pl
from jax.experimental.pallas import tpu as pltpu
from jax.experimental.pallas import tpu_sc as plsc


def _add_kernel(p_ref, e_ref, o_ref):
    o_ref[...] = p_ref[...] + e_ref[...][None, :, :]


def _make_gather(T, D, dtype):
    info = plsc.get_sparse_core_info()
    nc, ns = info.num_cores, info.num_subcores
    nw = nc * ns
    rows_per_w = T // nw
    chunk = 32
    n_chunks = rows_per_w // chunk
    mesh = plsc.VectorSubcoreMesh(core_axis_name="c", subcore_axis_name="s")

    @functools.partial(
        pl.kernel,
        mesh=mesh,
        out_type=jax.ShapeDtypeStruct((T, D), dtype),
        scratch_types=[
            pltpu.VMEM((chunk,), jnp.int32),
            pltpu.VMEM((chunk, D), dtype),
            pltpu.SemaphoreType.DMA,
        ],
    )
    def gather(table_hbm, idx_hbm, out_hbm, idx_v, rows_v, sem):
        wid = lax.axis_index("s") * nc + lax.axis_index("c")
        base = wid * rows_per_w
        for c in range(n_chunks):
            off = base + c * chunk
            pltpu.sync_copy(idx_hbm.at[pl.ds(off, chunk)], idx_v)
            pltpu.async_copy(table_hbm.at[idx_v], rows_v, sem).wait()
            pltpu.sync_copy(rows_v, out_hbm.at[pl.ds(off, chunk)])

    return gather


def kernel(patch_tensor, emb, stride):
    B, T, D = patch_tensor.shape
    E = emb.shape[0]
    offsets = jnp.clip(
        jnp.arange(T, dtype=jnp.int32) * stride + stride // 2, 0, E - 1
    ).astype(jnp.int32)
    pos_emb = _make_gather(T, D, emb.dtype)(emb, offsets)
    Tt = 1024
    grid = (T // Tt, B)
    return pl.pallas_call(
        _add_kernel,
        grid=grid,
        in_specs=[
            pl.BlockSpec((1, Tt, D), lambda i, b: (b, i, 0)),
            pl.BlockSpec((Tt, D), lambda i, b: (i, 0)),
        ],
        out_specs=pl.BlockSpec((1, Tt, D), lambda i, b: (b, i, 0)),
        out_shape=jax.ShapeDtypeStruct((B, T, D), patch_tensor.dtype),
    )(patch_tensor, pos_emb)


# speedup vs baseline: 1.0152x; 1.0152x over previous
import functools

import jax
import jax.numpy as jnp
from jax import lax
from jax.experimental import pallas as pl
from jax.experimental.pallas import tpu as pltpu
from jax.experimental.pallas import tpu_sc as plsc


def _add_kernel(p_ref, e_ref, o_ref):
    o_ref[...] = p_ref[...] + e_ref[...][None, :, :]


def _make_gather(T, D, dtype):
    info = plsc.get_sparse_core_info()
    nc, ns = info.num_cores, info.num_subcores
    nw = nc * ns
    rows_per_w = T // nw
    chunk = 32
    n_chunks = rows_per_w // chunk
    mesh = plsc.VectorSubcoreMesh(core_axis_name="c", subcore_axis_name="s")

    @functools.partial(
        pl.kernel,
        mesh=mesh,
        out_type=jax.ShapeDtypeStruct((T, D), dtype),
        scratch_types=[
            pltpu.VMEM((chunk,), jnp.int32),
            pltpu.VMEM((chunk, D), dtype),
            pltpu.SemaphoreType.DMA,
        ],
    )
    def gather(table_hbm, idx_hbm, out_hbm, idx_v, rows_v, sem):
        wid = lax.axis_index("s") * nc + lax.axis_index("c")
        base = wid * rows_per_w
        for c in range(n_chunks):
            off = base + c * chunk
            pltpu.sync_copy(idx_hbm.at[pl.ds(off, chunk)], idx_v)
            pltpu.async_copy(table_hbm.at[idx_v], rows_v, sem).wait()
            pltpu.sync_copy(rows_v, out_hbm.at[pl.ds(off, chunk)])

    return gather


def _add_kernel_acc(a_ref, p_ref, e_ref, o_ref):
    del a_ref
    o_ref[...] = p_ref[...] + e_ref[...][None, :, :]


def kernel(patch_tensor, emb, stride):
    B, T, D = patch_tensor.shape
    E = emb.shape[0]
    offsets = jnp.clip(
        jnp.arange(T, dtype=jnp.int32) * stride + stride // 2, 0, E - 1
    ).astype(jnp.int32)
    K = 4
    Tc = T // K
    gather = _make_gather(Tc, D, emb.dtype)
    pos = [gather(emb, offsets[k * Tc:(k + 1) * Tc]) for k in range(K)]
    Tt = 1024
    nI = Tc // Tt
    out = None
    for k in range(K):
        args = [patch_tensor, pos[k]]
        in_specs = [
            pl.BlockSpec((1, Tt, D), functools.partial(lambda k, i, b: (b, k * nI + i, 0), k)),
            pl.BlockSpec((Tt, D), lambda i, b: (i, 0)),
        ]
        aliases = {}
        if out is not None:
            args = [out] + args
            in_specs = [pl.BlockSpec((1, 8, 128), lambda i, b: (0, 0, 0))] + in_specs
            aliases = {0: 0}
        out = pl.pallas_call(
            _add_kernel_acc if out is not None else
            (lambda p_ref, e_ref, o_ref: _add_kernel(p_ref, e_ref, o_ref)),
            grid=(nI, B),
            in_specs=in_specs,
            out_specs=pl.BlockSpec(
                (1, Tt, D), functools.partial(lambda k, i, b: (b, k * nI + i, 0), k)),
            out_shape=jax.ShapeDtypeStruct((B, T, D), patch_tensor.dtype),
            input_output_aliases=aliases,
        )(*args)
    return out
